# baseline (speedup 1.0000x reference)
import jax
import jax.numpy as jnp
from jax.experimental import pallas as pl
from jax.experimental.pallas import tpu as pltpu

_BB = 5


def _add_kernel(x_ref, t_ref, o_ref):
    o_ref[...] = x_ref[...] + t_ref[...][None]


def kernel(x, table):
    batch, num_patches, dim = x.shape
    grid = (pl.cdiv(batch, _BB),)
    return pl.pallas_call(
        _add_kernel,
        grid=grid,
        in_specs=[
            pl.BlockSpec((_BB, num_patches, dim), lambda b: (b, 0, 0)),
            pl.BlockSpec((num_patches, dim), lambda b: (0, 0)),
        ],
        out_specs=pl.BlockSpec((_BB, num_patches, dim), lambda b: (b, 0, 0)),
        out_shape=jax.ShapeDtypeStruct(x.shape, x.dtype),
        compiler_params=pltpu.CompilerParams(
            dimension_semantics=("parallel",),
            vmem_limit_bytes=120 * 1024 * 1024,
        ),
    )(x, table)


# speedup vs baseline: 1.0023x; 1.0023x over previous
import jax
import jax.numpy as jnp
from jax import lax
from jax.experimental import pallas as pl
from jax.experimental.pallas import tpu as pltpu

_K = 5
_CB = 2


def _ring_kernel(x_hbm, t_hbm, o_hbm, tbuf, xbuf, obuf, in_sem, out_sem,
                 t_sem):
    n_steps = x_hbm.shape[0] // _CB

    def in_copy(s, k):
        return pltpu.make_async_copy(x_hbm.at[pl.ds(s * _CB, _CB)], xbuf.at[k], in_sem.at[k])

    def out_copy(s, k):
        return pltpu.make_async_copy(obuf.at[k], o_hbm.at[pl.ds(s * _CB, _CB)], out_sem.at[k])

    pltpu.make_async_copy(t_hbm, tbuf, t_sem).start()
    for s in range(_K):
        in_copy(s, s).start()
    pltpu.make_async_copy(t_hbm, tbuf, t_sem).wait()

    def body(s, _):
        k = lax.rem(s, _K)
        in_copy(s, k).wait()

        @pl.when(s >= _K)
        def _():
            out_copy(s - _K, k).wait()

        obuf[k] = xbuf[k] + tbuf[...][None]
        out_copy(s, k).start()

        @pl.when(s + _K < n_steps)
        def _():
            in_copy(s + _K, k).start()

        return 0

    lax.fori_loop(0, n_steps, body, 0)

    def drain(s, _):
        k = lax.rem(s, _K)
        out_copy(s, k).wait()
        return 0

    lax.fori_loop(n_steps - _K, n_steps, drain, 0)


def kernel(x, table):
    batch, num_patches, dim = x.shape
    x3 = x.reshape(batch, num_patches, dim)
    out = pl.pallas_call(
        _ring_kernel,
        in_specs=[
            pl.BlockSpec(memory_space=pltpu.MemorySpace.HBM),
            pl.BlockSpec(memory_space=pltpu.MemorySpace.HBM),
        ],
        out_specs=pl.BlockSpec(memory_space=pltpu.MemorySpace.HBM),
        out_shape=jax.ShapeDtypeStruct((batch, num_patches, dim), x.dtype),
        scratch_shapes=[
            pltpu.VMEM((num_patches, dim), x.dtype),
            pltpu.VMEM((_K, _CB, num_patches, dim), x.dtype),
            pltpu.VMEM((_K, _CB, num_patches, dim), x.dtype),
            pltpu.SemaphoreType.DMA((_K,)),
            pltpu.SemaphoreType.DMA((_K,)),
            pltpu.SemaphoreType.DMA,
        ],
        compiler_params=pltpu.CompilerParams(
            vmem_limit_bytes=120 * 1024 * 1024,
        ),
    )(x3, table)
    return out


# speedup vs baseline: 1.0026x; 1.0003x over previous
import jax
import jax.numpy as jnp
from jax import lax
from jax.experimental import pallas as pl
from jax.experimental.pallas import tpu as pltpu

_KI = 6
_KO = 4
_CB = 2


def _ring_kernel(x_hbm, t_hbm, o_hbm, tbuf, xbuf, obuf, in_sem, out_sem,
                 t_sem):
    n_steps = x_hbm.shape[0] // _CB

    def in_copy(s, k):
        return pltpu.make_async_copy(
            x_hbm.at[pl.ds(s * _CB, _CB)], xbuf.at[k], in_sem.at[k])

    def out_copy(s, k):
        return pltpu.make_async_copy(
            obuf.at[k], o_hbm.at[pl.ds(s * _CB, _CB)], out_sem.at[k])

    pltpu.make_async_copy(t_hbm, tbuf, t_sem).start()
    for s in range(_KI):
        in_copy(s, s).start()
    pltpu.make_async_copy(t_hbm, tbuf, t_sem).wait()

    def body(s, _):
        ki = lax.rem(s, _KI)
        ko = lax.rem(s, _KO)
        in_copy(s, ki).wait()

        @pl.when(s >= _KO)
        def _():
            out_copy(s - _KO, ko).wait()

        obuf[ko] = xbuf[ki] + tbuf[...][None]
        out_copy(s, ko).start()

        @pl.when(s + _KI < n_steps)
        def _():
            in_copy(s + _KI, ki).start()

        return 0

    lax.fori_loop(0, n_steps, body, 0)

    def drain(s, _):
        out_copy(s, lax.rem(s, _KO)).wait()
        return 0

    lax.fori_loop(n_steps - _KO, n_steps, drain, 0)


def kernel(x, table):
    batch, num_patches, dim = x.shape
    out = pl.pallas_call(
        _ring_kernel,
        in_specs=[
            pl.BlockSpec(memory_space=pltpu.MemorySpace.HBM),
            pl.BlockSpec(memory_space=pltpu.MemorySpace.HBM),
        ],
        out_specs=pl.BlockSpec(memory_space=pltpu.MemorySpace.HBM),
        out_shape=jax.ShapeDtypeStruct((batch, num_patches, dim), x.dtype),
        scratch_shapes=[
            pltpu.VMEM((num_patches, dim), x.dtype),
            pltpu.VMEM((_KI, _CB, num_patches, dim), x.dtype),
            pltpu.VMEM((_KO, _CB, num_patches, dim), x.dtype),
            pltpu.SemaphoreType.DMA((_KI,)),
            pltpu.SemaphoreType.DMA((_KO,)),
            pltpu.SemaphoreType.DMA,
        ],
        compiler_params=pltpu.CompilerParams(
            vmem_limit_bytes=120 * 1024 * 1024,
        ),
    )(x, table)
    return out
